# Initial kernel scaffold; baseline (speedup 1.0000x reference)
import jax
import jax.numpy as jnp
from jax import lax
from jax.experimental import pallas as pl
from jax.experimental.pallas import tpu as pltpu
from jax.experimental.pallas import tpu_sc as plsc

N = 10000
E = 320000
D = 128
NC = 2
NS = 16
NW = NC * NS
EPW = E // NW
CHUNK = 80
NCHUNK = EPW // CHUNK
RPS = N // NS
ZROWS = 125

_MESH = plsc.VectorSubcoreMesh(
    core_axis_name="c", subcore_axis_name="s", num_cores=NC, num_subcores=NS
)

_HIGH = jax.lax.Precision.HIGHEST


def _deg_body(dst_hbm, out_hbm, dst_v, ones_v, zed_v, acc_sh, sem):
    c = lax.axis_index("c")
    s = lax.axis_index("s")
    wid = c * NS + s

    @pl.loop(0, CHUNK)
    def _(i):
        ones_v[i, :] = jnp.ones((16,), jnp.float32)

    @pl.loop(0, RPS)
    def _(i):
        zed_v[i, :] = jnp.zeros((16,), jnp.float32)

    pltpu.sync_copy(zed_v, acc_sh.at[pl.ds(s * RPS, RPS)])
    pltpu.sync_copy(dst_hbm.at[wid], dst_v)
    plsc.subcore_barrier()

    @pl.loop(0, NCHUNK)
    def _(ci):
        pltpu.sync_copy(ones_v, acc_sh.at[dst_v.at[ci]], add=True)

    plsc.subcore_barrier()
    pltpu.sync_copy(acc_sh.at[pl.ds(s * RPS, RPS)],
                    out_hbm.at[c, pl.ds(s * RPS, RPS)])


def _deg_pass(dst3):
    k = pl.kernel(
        _deg_body,
        out_type=jax.ShapeDtypeStruct((NC, N, 16), jnp.float32),
        mesh=_MESH,
        scratch_types=[
            pltpu.VMEM((NCHUNK, CHUNK), jnp.int32),
            pltpu.VMEM((CHUNK, 16), jnp.float32),
            pltpu.VMEM((RPS, 16), jnp.float32),
            pltpu.VMEM_SHARED((N, 16), jnp.float32),
            pltpu.SemaphoreType.DMA,
        ],
    )
    return k(dst3)


def _seg_body(u_hbm, src_hbm, dst_hbm, out_hbm,
              src_v, dst_v, rows_v, zacc_v, acc_sh, sem):
    c = lax.axis_index("c")
    s = lax.axis_index("s")
    wid = c * NS + s

    @pl.loop(0, ZROWS)
    def _(i):
        @pl.loop(0, D // 16)
        def _(j):
            zacc_v[i, pl.ds(j * 16, 16)] = jnp.zeros((16,), jnp.float32)

    @pl.loop(0, RPS // ZROWS)
    def _(k):
        pltpu.sync_copy(zacc_v, acc_sh.at[pl.ds(s * RPS + k * ZROWS, ZROWS)])

    pltpu.sync_copy(src_hbm.at[wid], src_v)
    pltpu.sync_copy(dst_hbm.at[wid], dst_v)
    plsc.subcore_barrier()

    @pl.loop(0, NCHUNK)
    def _(ci):
        pltpu.async_copy(u_hbm.at[src_v.at[ci]], rows_v, sem).wait()
        pltpu.sync_copy(rows_v, acc_sh.at[dst_v.at[ci]], add=True)

    plsc.subcore_barrier()
    pltpu.sync_copy(acc_sh.at[pl.ds(s * RPS, RPS)],
                    out_hbm.at[c, pl.ds(s * RPS, RPS)])


def _seg_pass(u, src3, dst3):
    k = pl.kernel(
        _seg_body,
        out_type=jax.ShapeDtypeStruct((NC, N, D), jnp.float32),
        mesh=_MESH,
        scratch_types=[
            pltpu.VMEM((NCHUNK, CHUNK), jnp.int32),
            pltpu.VMEM((NCHUNK, CHUNK), jnp.int32),
            pltpu.VMEM((CHUNK, D), jnp.float32),
            pltpu.VMEM((ZROWS, D), jnp.float32),
            pltpu.VMEM_SHARED((N, D), jnp.float32),
            pltpu.SemaphoreType.DMA,
        ],
    )
    return k(u, src3, dst3)


def _mm_body(x_ref, w_ref, o_ref):
    o_ref[...] = jnp.dot(x_ref[...], w_ref[...],
                         preferred_element_type=jnp.float32, precision=_HIGH)


def _matmul(x, w):
    return pl.pallas_call(
        _mm_body,
        out_shape=jax.ShapeDtypeStruct((x.shape[0], w.shape[1]), jnp.float32),
    )(x, w)


def _norm_u_body(cnt_ref, xw_ref, norm_ref, u_ref):
    deg = cnt_ref[0] + cnt_ref[1] + 1.0
    norm16 = lax.rsqrt(jnp.maximum(deg, 1.0))
    norm_ref[...] = norm16
    u_ref[...] = xw_ref[...] * norm16[:, 0:1]


def _norm_u(cnt, xw):
    return pl.pallas_call(
        _norm_u_body,
        out_shape=(
            jax.ShapeDtypeStruct((N, 16), jnp.float32),
            jax.ShapeDtypeStruct((N, D), jnp.float32),
        ),
    )(cnt, xw)


def _mid_body(norm_ref, s_ref, u_ref, b_ref, w_ref, u2_ref):
    norm1 = norm_ref[:, 0:1]
    agg = (s_ref[0] + s_ref[1] + u_ref[...]) * norm1 + b_ref[...]
    h = jnp.maximum(agg, 0.0)
    xw2 = jnp.dot(h, w_ref[...],
                  preferred_element_type=jnp.float32, precision=_HIGH)
    u2_ref[...] = xw2 * norm1


def _mid(norm16, s_partials, u1, b1, w2):
    return pl.pallas_call(
        _mid_body,
        out_shape=jax.ShapeDtypeStruct((N, D), jnp.float32),
    )(norm16, s_partials, u1, b1, w2)


def _final_body(norm_ref, s_ref, u_ref, b_ref, z_ref):
    norm1 = norm_ref[:, 0:1]
    z_ref[...] = (s_ref[0] + s_ref[1] + u_ref[...]) * norm1 + b_ref[...]


def _final(norm16, s_partials, u2, b2):
    return pl.pallas_call(
        _final_body,
        out_shape=jax.ShapeDtypeStruct((N, D), jnp.float32),
    )(norm16, s_partials, u2, b2)


@jax.jit
def kernel(x, edge_index, W1, b1, W2, b2):
    src3 = edge_index[0].reshape(NW, NCHUNK, CHUNK)
    dst3 = edge_index[1].reshape(NW, NCHUNK, CHUNK)
    b1r = b1.reshape(1, D)
    b2r = b2.reshape(1, D)

    cnt = _deg_pass(dst3)
    xw1 = _matmul(x, W1)
    norm16, u1 = _norm_u(cnt, xw1)
    s1 = _seg_pass(u1, src3, dst3)
    u2 = _mid(norm16, s1, u1, b1r, W2)
    s2 = _seg_pass(u2, src3, dst3)
    return _final(norm16, s2, u2, b2r)


# speedup vs baseline: 8.9004x; 8.9004x over previous
import jax
import jax.numpy as jnp
from jax import lax
from jax.experimental import pallas as pl
from jax.experimental.pallas import tpu as pltpu
from jax.experimental.pallas import tpu_sc as plsc

N = 10000
E = 320000
D = 128
NC = 2
NS = 16
NW = NC * NS
EPW = 10240
CHUNK = 80
NCHUNK = EPW // CHUNK
GSZ = 8
NGRP = NCHUNK // GSZ
PADN = 10240
RPS = PADN // NS
NZ = RPS // D

_MESH = plsc.VectorSubcoreMesh(
    core_axis_name="c", subcore_axis_name="s", num_cores=NC, num_subcores=NS
)

_HIGH = jax.lax.Precision.HIGHEST


def _fill_iota_rows(idx_v, s):
    @pl.loop(0, NZ)
    def _(k):
        @pl.loop(0, 8)
        def _(j):
            idx_v[k, pl.ds(j * 16, 16)] = (
                lax.iota(jnp.int32, 16) + (s * RPS + k * D + j * 16))


def _zero_acc(idx_v, zed_v, acc_sh):
    @pl.loop(0, D)
    def _(i):
        @pl.loop(0, 8)
        def _(j):
            zed_v[i, pl.ds(j * 16, 16)] = jnp.zeros((16,), jnp.float32)

    @pl.loop(0, NZ)
    def _(k):
        pltpu.sync_copy(zed_v, acc_sh.at[idx_v.at[k]])


def _drain_acc(idx_v, rd_v, acc_sh, out_hbm, c, s):
    @pl.loop(0, NZ)
    def _(k):
        pltpu.sync_copy(acc_sh.at[idx_v.at[k]], rd_v)
        pltpu.sync_copy(rd_v, out_hbm.at[c, pl.ds(s * RPS + k * D, D)])


def _deg_body(dst_hbm, out_hbm, dst_v, ones_v, idx_v, zed_v, acc_sh, sem):
    c = lax.axis_index("c")
    s = lax.axis_index("s")
    wid = c * NS + s

    _fill_iota_rows(idx_v, s)

    @pl.loop(0, CHUNK)
    def _(i):
        @pl.loop(0, 8)
        def _(j):
            ones_v[i, pl.ds(j * 16, 16)] = jnp.ones((16,), jnp.float32)

    _zero_acc(idx_v, zed_v, acc_sh)
    plsc.subcore_barrier()

    @pl.loop(0, NGRP)
    def _(g):
        pltpu.sync_copy(dst_hbm.at[wid, pl.ds(g * GSZ, GSZ)], dst_v)

        @pl.loop(0, GSZ)
        def _(ci):
            pltpu.sync_copy(ones_v, acc_sh.at[dst_v.at[ci]], add=True)

    plsc.subcore_barrier()
    _drain_acc(idx_v, zed_v, acc_sh, out_hbm, c, s)


def _deg_pass(dst3):
    k = pl.kernel(
        _deg_body,
        out_type=jax.ShapeDtypeStruct((NC, PADN, D), jnp.float32),
        mesh=_MESH,
        scratch_types=[
            pltpu.VMEM((GSZ, CHUNK), jnp.int32),
            pltpu.VMEM((CHUNK, D), jnp.float32),
            pltpu.VMEM((NZ, D), jnp.int32),
            pltpu.VMEM((D, D), jnp.float32),
            pltpu.VMEM_SHARED((PADN, D), jnp.float32),
            pltpu.SemaphoreType.DMA,
        ],
    )
    return k(dst3)


def _seg_body(u_hbm, src_hbm, dst_hbm, out_hbm,
              src_v, dst_v, rows_v, idx_v, zed_v, acc_sh, sem):
    c = lax.axis_index("c")
    s = lax.axis_index("s")
    wid = c * NS + s

    _fill_iota_rows(idx_v, s)
    _zero_acc(idx_v, zed_v, acc_sh)
    plsc.subcore_barrier()

    @pl.loop(0, NGRP)
    def _(g):
        pltpu.sync_copy(src_hbm.at[wid, pl.ds(g * GSZ, GSZ)], src_v)
        pltpu.sync_copy(dst_hbm.at[wid, pl.ds(g * GSZ, GSZ)], dst_v)

        @pl.loop(0, GSZ)
        def _(ci):
            pltpu.async_copy(u_hbm.at[src_v.at[ci]], rows_v, sem).wait()
            pltpu.sync_copy(rows_v, acc_sh.at[dst_v.at[ci]], add=True)

    plsc.subcore_barrier()
    _drain_acc(idx_v, zed_v, acc_sh, out_hbm, c, s)


def _seg_pass(u, src3, dst3):
    k = pl.kernel(
        _seg_body,
        out_type=jax.ShapeDtypeStruct((NC, PADN, D), jnp.float32),
        mesh=_MESH,
        scratch_types=[
            pltpu.VMEM((GSZ, CHUNK), jnp.int32),
            pltpu.VMEM((GSZ, CHUNK), jnp.int32),
            pltpu.VMEM((CHUNK, D), jnp.float32),
            pltpu.VMEM((NZ, D), jnp.int32),
            pltpu.VMEM((D, D), jnp.float32),
            pltpu.VMEM_SHARED((PADN, D), jnp.float32),
            pltpu.SemaphoreType.DMA,
        ],
    )
    return k(u, src3, dst3)


def _mm_body(x_ref, w_ref, o_ref):
    o_ref[...] = jnp.dot(x_ref[...], w_ref[...],
                         preferred_element_type=jnp.float32, precision=_HIGH)


def _matmul(x, w):
    return pl.pallas_call(
        _mm_body,
        out_shape=jax.ShapeDtypeStruct((x.shape[0], w.shape[1]), jnp.float32),
    )(x, w)


def _norm_u_body(cnt_ref, xw_ref, norm_ref, u_ref):
    deg = cnt_ref[0, :N] + cnt_ref[1, :N] + 1.0
    norm = lax.rsqrt(jnp.maximum(deg, 1.0))
    norm_ref[...] = norm
    u_ref[...] = xw_ref[...] * norm


def _norm_u(cnt, xw):
    return pl.pallas_call(
        _norm_u_body,
        out_shape=(
            jax.ShapeDtypeStruct((N, D), jnp.float32),
            jax.ShapeDtypeStruct((N, D), jnp.float32),
        ),
    )(cnt, xw)


def _mid_body(norm_ref, s_ref, u_ref, b_ref, w_ref, u2_ref):
    agg = ((s_ref[0, :N] + s_ref[1, :N] + u_ref[...]) * norm_ref[...]
           + b_ref[...])
    h = jnp.maximum(agg, 0.0)
    xw2 = jnp.dot(h, w_ref[...],
                  preferred_element_type=jnp.float32, precision=_HIGH)
    u2_ref[...] = xw2 * norm_ref[...]


def _mid(norm, s_partials, u1, b1, w2):
    return pl.pallas_call(
        _mid_body,
        out_shape=jax.ShapeDtypeStruct((N, D), jnp.float32),
    )(norm, s_partials, u1, b1, w2)


def _final_body(norm_ref, s_ref, u_ref, b_ref, z_ref):
    z_ref[...] = ((s_ref[0, :N] + s_ref[1, :N] + u_ref[...]) * norm_ref[...]
                  + b_ref[...])


def _final(norm, s_partials, u2, b2):
    return pl.pallas_call(
        _final_body,
        out_shape=jax.ShapeDtypeStruct((N, D), jnp.float32),
    )(norm, s_partials, u2, b2)


@jax.jit
def kernel(x, edge_index, W1, b1, W2, b2):
    src2 = jnp.pad(edge_index[0].reshape(NW, E // NW), ((0, 0), (0, EPW - E // NW)))
    dst2 = jnp.pad(edge_index[1].reshape(NW, E // NW), ((0, 0), (0, EPW - E // NW)),
                   constant_values=0)
    junk = (N + 200 + jnp.arange(NW, dtype=edge_index.dtype))[:, None]
    dst2 = dst2.at[:, E // NW:].set(jnp.broadcast_to(junk, (NW, EPW - E // NW)))
    src3 = src2.reshape(NW, NCHUNK, CHUNK)
    dst3 = dst2.reshape(NW, NCHUNK, CHUNK)
    b1r = b1.reshape(1, D)
    b2r = b2.reshape(1, D)

    cnt = _deg_pass(dst3)
    xw1 = _matmul(x, W1)
    norm, u1 = _norm_u(cnt, xw1)
    s1 = _seg_pass(u1, src3, dst3)
    u2 = _mid(norm, s1, u1, b1r, W2)
    s2 = _seg_pass(u2, src3, dst3)
    return _final(norm, s2, u2, b2r)


# speedup vs baseline: 9.8661x; 1.1085x over previous
import jax
import jax.numpy as jnp
from jax import lax
from jax.experimental import pallas as pl
from jax.experimental.pallas import tpu as pltpu
from jax.experimental.pallas import tpu_sc as plsc

N = 10000
E = 320000
D = 128
NC = 2
NS = 16
NW = NC * NS
EPW = 10240
CHUNK = 80
NCHUNK = EPW // CHUNK
GSZ = 16
NGRP = NCHUNK // GSZ
PADN = 10240
RPS = PADN // NS
NZ = RPS // D

_MESH = plsc.VectorSubcoreMesh(
    core_axis_name="c", subcore_axis_name="s", num_cores=NC, num_subcores=NS
)

_HIGH = jax.lax.Precision.HIGHEST


def _fill_iota_rows(idx_v, s):
    @pl.loop(0, NZ)
    def _(k):
        @pl.loop(0, 8)
        def _(j):
            idx_v[k, pl.ds(j * 16, 16)] = (
                lax.iota(jnp.int32, 16) + (s * RPS + k * D + j * 16))


def _zero_acc(idx_v, zed_v, acc_sh):
    @pl.loop(0, D)
    def _(i):
        @pl.loop(0, 8)
        def _(j):
            zed_v[i, pl.ds(j * 16, 16)] = jnp.zeros((16,), jnp.float32)

    @pl.loop(0, NZ)
    def _(k):
        pltpu.sync_copy(zed_v, acc_sh.at[idx_v.at[k]])


def _drain_acc(idx_v, rd_v, acc_sh, out_hbm, c, s):
    @pl.loop(0, NZ)
    def _(k):
        pltpu.sync_copy(acc_sh.at[idx_v.at[k]], rd_v)
        pltpu.sync_copy(rd_v, out_hbm.at[c, pl.ds(s * RPS + k * D, D)])


def _deg_body(dst_hbm, out_hbm, dst_v, ones_v, idx_v, zed_v, acc_sh, sem):
    c = lax.axis_index("c")
    s = lax.axis_index("s")
    wid = c * NS + s

    _fill_iota_rows(idx_v, s)

    @pl.loop(0, CHUNK)
    def _(i):
        @pl.loop(0, 8)
        def _(j):
            ones_v[i, pl.ds(j * 16, 16)] = jnp.ones((16,), jnp.float32)

    _zero_acc(idx_v, zed_v, acc_sh)
    plsc.subcore_barrier()

    @pl.loop(0, NGRP)
    def _(g):
        pltpu.sync_copy(dst_hbm.at[wid, pl.ds(g * GSZ, GSZ)], dst_v)

        @pl.loop(0, GSZ)
        def _(ci):
            pltpu.sync_copy(ones_v, acc_sh.at[dst_v.at[ci]], add=True)

    plsc.subcore_barrier()
    _drain_acc(idx_v, zed_v, acc_sh, out_hbm, c, s)


def _deg_pass(dst3):
    k = pl.kernel(
        _deg_body,
        out_type=jax.ShapeDtypeStruct((NC, PADN, D), jnp.float32),
        mesh=_MESH,
        scratch_types=[
            pltpu.VMEM((GSZ, CHUNK), jnp.int32),
            pltpu.VMEM((CHUNK, D), jnp.float32),
            pltpu.VMEM((NZ, D), jnp.int32),
            pltpu.VMEM((D, D), jnp.float32),
            pltpu.VMEM_SHARED((PADN, D), jnp.float32),
            pltpu.SemaphoreType.DMA,
        ],
    )
    return k(dst3)


def _seg_body(u_hbm, src_hbm, dst_hbm, out_hbm,
              src_v, dst_v, rows_a, rows_b, idx_v, zed_v, acc_sh,
              sem_a, sem_b):
    c = lax.axis_index("c")
    s = lax.axis_index("s")
    wid = c * NS + s

    _fill_iota_rows(idx_v, s)
    _zero_acc(idx_v, zed_v, acc_sh)
    plsc.subcore_barrier()

    @pl.loop(0, NGRP)
    def _(g):
        pltpu.sync_copy(src_hbm.at[wid, pl.ds(g * GSZ, GSZ)], src_v)
        pltpu.sync_copy(dst_hbm.at[wid, pl.ds(g * GSZ, GSZ)], dst_v)
        pltpu.make_async_copy(u_hbm.at[src_v.at[0]], rows_a, sem_a).start()

        @pl.loop(0, GSZ // 2)
        def _(h):
            ca = 2 * h
            cb = 2 * h + 1
            pltpu.make_async_copy(u_hbm.at[src_v.at[ca]], rows_a, sem_a).wait()
            pltpu.make_async_copy(u_hbm.at[src_v.at[cb]], rows_b, sem_b).start()
            pltpu.sync_copy(rows_a, acc_sh.at[dst_v.at[ca]], add=True)
            pltpu.make_async_copy(u_hbm.at[src_v.at[cb]], rows_b, sem_b).wait()

            @pl.when(h < GSZ // 2 - 1)
            def _():
                pltpu.make_async_copy(u_hbm.at[src_v.at[ca + 2]],
                                      rows_a, sem_a).start()

            pltpu.sync_copy(rows_b, acc_sh.at[dst_v.at[cb]], add=True)

    plsc.subcore_barrier()
    _drain_acc(idx_v, zed_v, acc_sh, out_hbm, c, s)


def _seg_pass(u, src3, dst3):
    k = pl.kernel(
        _seg_body,
        out_type=jax.ShapeDtypeStruct((NC, PADN, D), jnp.float32),
        mesh=_MESH,
        scratch_types=[
            pltpu.VMEM((GSZ, CHUNK), jnp.int32),
            pltpu.VMEM((GSZ, CHUNK), jnp.int32),
            pltpu.VMEM((CHUNK, D), jnp.float32),
            pltpu.VMEM((CHUNK, D), jnp.float32),
            pltpu.VMEM((NZ, D), jnp.int32),
            pltpu.VMEM((D, D), jnp.float32),
            pltpu.VMEM_SHARED((PADN, D), jnp.float32),
            pltpu.SemaphoreType.DMA,
            pltpu.SemaphoreType.DMA,
        ],
    )
    return k(u, src3, dst3)


def _mm_body(x_ref, w_ref, o_ref):
    o_ref[...] = jnp.dot(x_ref[...], w_ref[...],
                         preferred_element_type=jnp.float32, precision=_HIGH)


def _matmul(x, w):
    return pl.pallas_call(
        _mm_body,
        out_shape=jax.ShapeDtypeStruct((x.shape[0], w.shape[1]), jnp.float32),
    )(x, w)


def _norm_u_body(cnt_ref, xw_ref, norm_ref, u_ref):
    deg = cnt_ref[0, :N] + cnt_ref[1, :N] + 1.0
    norm = lax.rsqrt(jnp.maximum(deg, 1.0))
    norm_ref[...] = norm
    u_ref[...] = xw_ref[...] * norm


def _norm_u(cnt, xw):
    return pl.pallas_call(
        _norm_u_body,
        out_shape=(
            jax.ShapeDtypeStruct((N, D), jnp.float32),
            jax.ShapeDtypeStruct((N, D), jnp.float32),
        ),
    )(cnt, xw)


def _mid_body(norm_ref, s_ref, u_ref, b_ref, w_ref, u2_ref):
    agg = ((s_ref[0, :N] + s_ref[1, :N] + u_ref[...]) * norm_ref[...]
           + b_ref[...])
    h = jnp.maximum(agg, 0.0)
    xw2 = jnp.dot(h, w_ref[...],
                  preferred_element_type=jnp.float32, precision=_HIGH)
    u2_ref[...] = xw2 * norm_ref[...]


def _mid(norm, s_partials, u1, b1, w2):
    return pl.pallas_call(
        _mid_body,
        out_shape=jax.ShapeDtypeStruct((N, D), jnp.float32),
    )(norm, s_partials, u1, b1, w2)


def _final_body(norm_ref, s_ref, u_ref, b_ref, z_ref):
    z_ref[...] = ((s_ref[0, :N] + s_ref[1, :N] + u_ref[...]) * norm_ref[...]
                  + b_ref[...])


def _final(norm, s_partials, u2, b2):
    return pl.pallas_call(
        _final_body,
        out_shape=jax.ShapeDtypeStruct((N, D), jnp.float32),
    )(norm, s_partials, u2, b2)


@jax.jit
def kernel(x, edge_index, W1, b1, W2, b2):
    src2 = jnp.pad(edge_index[0].reshape(NW, E // NW), ((0, 0), (0, EPW - E // NW)))
    dst2 = jnp.pad(edge_index[1].reshape(NW, E // NW), ((0, 0), (0, EPW - E // NW)),
                   constant_values=0)
    junk = (N + 200 + jnp.arange(NW, dtype=edge_index.dtype))[:, None]
    dst2 = dst2.at[:, E // NW:].set(jnp.broadcast_to(junk, (NW, EPW - E // NW)))
    src3 = src2.reshape(NW, NCHUNK, CHUNK)
    dst3 = dst2.reshape(NW, NCHUNK, CHUNK)
    b1r = b1.reshape(1, D)
    b2r = b2.reshape(1, D)

    cnt = _deg_pass(dst3)
    xw1 = _matmul(x, W1)
    norm, u1 = _norm_u(cnt, xw1)
    s1 = _seg_pass(u1, src3, dst3)
    u2 = _mid(norm, s1, u1, b1r, W2)
    s2 = _seg_pass(u2, src3, dst3)
    return _final(norm, s2, u2, b2r)
